# baseline (speedup 1.0000x reference)
import jax
import jax.numpy as jnp
from jax.experimental import pallas as pl

_EDGE_SP = 512
_FEAT_SP = 256


def _vmask_kernel(sar_ref, out_ref):
    out_ref[...] = jnp.sum(sar_ref[...], axis=1)


def _cos_kernel(sa_ref, st_ref, ra_ref, rt_ref, rgb_out, sar_out):
    def cs(a, t):
        num = jax.lax.dot_general(
            a, t, (((1,), (1,)), ((), ())),
            precision=jax.lax.Precision.HIGHEST,
            preferred_element_type=jnp.float32,
        )
        na = jnp.sqrt(jnp.sum(a * a, axis=1, keepdims=True))
        nt = jnp.sqrt(jnp.sum(t * t, axis=1, keepdims=True))
        den = na * nt.T
        return num / jnp.maximum(den, 1e-8)

    rgb_out[...] = cs(ra_ref[...], rt_ref[...])
    sar_out[...] = cs(sa_ref[...], st_ref[...])


def _resize_bilinear(x, out_h, out_w):
    N, C, H, W = x.shape
    ys = jnp.linspace(0.0, H - 1, out_h)
    xs = jnp.linspace(0.0, W - 1, out_w)
    y0 = jnp.floor(ys).astype(jnp.int32)
    y1 = jnp.minimum(y0 + 1, H - 1)
    wy = ys - y0.astype(jnp.float32)
    x0 = jnp.floor(xs).astype(jnp.int32)
    x1 = jnp.minimum(x0 + 1, W - 1)
    wx = xs - x0.astype(jnp.float32)
    rows0 = x[:, :, y0, :]
    rows1 = x[:, :, y1, :]
    top = rows0[:, :, :, x0] * (1.0 - wx)[None, None, None, :] + rows0[:, :, :, x1] * wx[None, None, None, :]
    bot = rows1[:, :, :, x0] * (1.0 - wx)[None, None, None, :] + rows1[:, :, :, x1] * wx[None, None, None, :]
    return top * (1.0 - wy)[None, None, :, None] + bot * wy[None, None, :, None]


def _gumbel_topk(key, weights, k):
    g = jax.random.gumbel(key, weights.shape, dtype=jnp.float32)
    logits = jnp.where(weights > 0, jnp.log(jnp.maximum(weights, 1e-30)), -jnp.inf) + g
    _, idx = jax.lax.top_k(logits, k)
    return idx


def kernel(RGB_feat, SAR_feat, edge_mask):
    N, C, H, W = SAR_feat.shape
    P = N * H * W

    vm = pl.pallas_call(
        _vmask_kernel,
        grid=(N, H // 8),
        in_specs=[pl.BlockSpec((1, C, 8, W), lambda n, h: (n, 0, h, 0))],
        out_specs=pl.BlockSpec((1, 8, W), lambda n, h: (n, h, 0)),
        out_shape=jax.ShapeDtypeStruct((N, H, W), jnp.float32),
    )(SAR_feat)
    valid_mask = (vm.reshape(-1) != 0).astype(jnp.float32)
    feat_weight = valid_mask

    skey = jax.random.key(42)
    kf, ke = jax.random.split(skey)
    feat_index = _gumbel_topk(kf, feat_weight, _FEAT_SP)

    em = _resize_bilinear(edge_mask, H, W)
    em = (em > 0.5).astype(jnp.float32)
    em_flat = jnp.transpose(em, (0, 2, 3, 1)).reshape(-1) * valid_mask
    edge_weight = feat_weight * em_flat
    edge_index = _gumbel_topk(ke, edge_weight, _EDGE_SP)

    sar3 = SAR_feat.reshape(N, C, H * W)
    rgb3 = RGB_feat.reshape(N, C, H * W)

    def rows(feats3, idx):
        n_idx = idx // (H * W)
        hw_idx = idx % (H * W)
        return feats3[n_idx, :, hw_idx]

    SAR_anch = rows(sar3, edge_index)
    RGB_anch = rows(rgb3, edge_index)
    SAR_tar = rows(sar3, feat_index)
    RGB_tar = rows(rgb3, feat_index)

    RGB_mat, SAR_mat = pl.pallas_call(
        _cos_kernel,
        out_shape=(
            jax.ShapeDtypeStruct((_EDGE_SP, _FEAT_SP), jnp.float32),
            jax.ShapeDtypeStruct((_EDGE_SP, _FEAT_SP), jnp.float32),
        ),
    )(SAR_anch, SAR_tar, RGB_anch, RGB_tar)
    return (RGB_mat, SAR_mat)


# speedup vs baseline: 1.2427x; 1.2427x over previous
import functools

import jax
import jax.numpy as jnp
from jax import lax
from jax.experimental import pallas as pl
from jax.experimental.pallas import tpu as pltpu
from jax.experimental.pallas import tpu_sc as plsc

_EDGE_SP = 512
_FEAT_SP = 256
_NW = 32
_K = _EDGE_SP + _FEAT_SP
_PPW = _K // _NW
_CH = 192
_CHUNKS = _PPW * _CH // 128


def _sc_gather_body(sar_hbm, rgb_hbm, idx_hbm, sar_out, rgb_out,
                    idx_v, bs_v, br_v, sem_s, sem_r):
    cid = lax.axis_index("c")
    sid = lax.axis_index("s")
    wid = sid * 2 + cid
    pltpu.sync_copy(idx_hbm.at[wid], idx_v)

    def fire(j, carry):
        pltpu.async_copy(sar_hbm.at[idx_v.at[j]], bs_v.at[j], sem_s)
        pltpu.async_copy(rgb_hbm.at[idx_v.at[j]], br_v.at[j], sem_r)
        return carry

    lax.fori_loop(0, _CHUNKS, fire, 0)
    pltpu.make_async_copy(sar_out.at[wid], bs_v, sem_s).wait()
    pltpu.make_async_copy(rgb_out.at[wid], br_v, sem_r).wait()
    pltpu.sync_copy(bs_v, sar_out.at[wid])
    pltpu.sync_copy(br_v, rgb_out.at[wid])


def _vmask_kernel(sar_ref, out_ref):
    out_ref[...] = jnp.sum(sar_ref[...], axis=1)


def _cos_kernel(sa_ref, st_ref, ra_ref, rt_ref, rgb_out, sar_out):
    def cs(a, t):
        num = jax.lax.dot_general(
            a, t, (((1,), (1,)), ((), ())),
            precision=jax.lax.Precision.HIGHEST,
            preferred_element_type=jnp.float32,
        )
        na = jnp.sqrt(jnp.sum(a * a, axis=1, keepdims=True))
        nt = jnp.sqrt(jnp.sum(t * t, axis=1, keepdims=True))
        den = na * nt.T
        return num / jnp.maximum(den, 1e-8)

    rgb_out[...] = cs(ra_ref[...], rt_ref[...])
    sar_out[...] = cs(sa_ref[...], st_ref[...])


def _resize_bilinear(x, out_h, out_w):
    N, C, H, W = x.shape
    ys = jnp.linspace(0.0, H - 1, out_h)
    xs = jnp.linspace(0.0, W - 1, out_w)
    y0 = jnp.floor(ys).astype(jnp.int32)
    y1 = jnp.minimum(y0 + 1, H - 1)
    wy = ys - y0.astype(jnp.float32)
    x0 = jnp.floor(xs).astype(jnp.int32)
    x1 = jnp.minimum(x0 + 1, W - 1)
    wx = xs - x0.astype(jnp.float32)
    rows0 = x[:, :, y0, :]
    rows1 = x[:, :, y1, :]
    top = rows0[:, :, :, x0] * (1.0 - wx)[None, None, None, :] + rows0[:, :, :, x1] * wx[None, None, None, :]
    bot = rows1[:, :, :, x0] * (1.0 - wx)[None, None, None, :] + rows1[:, :, :, x1] * wx[None, None, None, :]
    return top * (1.0 - wy)[None, None, :, None] + bot * wy[None, None, :, None]


def _gumbel_topk(key, weights, k):
    g = jax.random.gumbel(key, weights.shape, dtype=jnp.float32)
    logits = jnp.where(weights > 0, jnp.log(jnp.maximum(weights, 1e-30)), -jnp.inf) + g
    _, idx = jax.lax.top_k(logits, k)
    return idx


def kernel(RGB_feat, SAR_feat, edge_mask):
    N, C, H, W = SAR_feat.shape
    P = N * H * W

    vm = pl.pallas_call(
        _vmask_kernel,
        grid=(N, H // 8),
        in_specs=[pl.BlockSpec((1, C, 8, W), lambda n, h: (n, 0, h, 0))],
        out_specs=pl.BlockSpec((1, 8, W), lambda n, h: (n, h, 0)),
        out_shape=jax.ShapeDtypeStruct((N, H, W), jnp.float32),
    )(SAR_feat)
    valid_mask = (vm.reshape(-1) != 0).astype(jnp.float32)
    feat_weight = valid_mask

    skey = jax.random.key(42)
    kf, ke = jax.random.split(skey)
    feat_index = _gumbel_topk(kf, feat_weight, _FEAT_SP)

    em = _resize_bilinear(edge_mask, H, W)
    em = (em > 0.5).astype(jnp.float32)
    em_flat = jnp.transpose(em, (0, 2, 3, 1)).reshape(-1) * valid_mask
    edge_weight = feat_weight * em_flat
    edge_index = _gumbel_topk(ke, edge_weight, _EDGE_SP)

    all_idx = jnp.concatenate([edge_index, feat_index])
    n_i = all_idx // (H * W)
    hw_i = all_idx % (H * W)
    word_base = n_i * (C * H * W) + hw_i
    widx = word_base[:, None] + (jnp.arange(C, dtype=jnp.int32) * (H * W))[None, :]
    widx = widx.reshape(_NW, _CHUNKS, 128)

    sc_gather = functools.partial(
        pl.kernel,
        out_type=(
            jax.ShapeDtypeStruct((_NW, _CHUNKS, 128), jnp.float32),
            jax.ShapeDtypeStruct((_NW, _CHUNKS, 128), jnp.float32),
        ),
        mesh=plsc.VectorSubcoreMesh(core_axis_name="c", subcore_axis_name="s"),
        scratch_types=[
            pltpu.VMEM((_CHUNKS, 128), jnp.int32),
            pltpu.VMEM((_CHUNKS, 128), jnp.float32),
            pltpu.VMEM((_CHUNKS, 128), jnp.float32),
            pltpu.SemaphoreType.DMA,
            pltpu.SemaphoreType.DMA,
        ],
    )(_sc_gather_body)
    sar_rows, rgb_rows = sc_gather(SAR_feat.reshape(-1), RGB_feat.reshape(-1), widx)
    sar_rows = sar_rows.reshape(_K, C)
    rgb_rows = rgb_rows.reshape(_K, C)
    SAR_anch, SAR_tar = sar_rows[:_EDGE_SP], sar_rows[_EDGE_SP:]
    RGB_anch, RGB_tar = rgb_rows[:_EDGE_SP], rgb_rows[_EDGE_SP:]

    RGB_mat, SAR_mat = pl.pallas_call(
        _cos_kernel,
        out_shape=(
            jax.ShapeDtypeStruct((_EDGE_SP, _FEAT_SP), jnp.float32),
            jax.ShapeDtypeStruct((_EDGE_SP, _FEAT_SP), jnp.float32),
        ),
    )(SAR_anch, SAR_tar, RGB_anch, RGB_tar)
    return (RGB_mat, SAR_mat)


# speedup vs baseline: 1.4031x; 1.1291x over previous
import functools

import jax
import jax.numpy as jnp
from jax import lax
from jax.experimental import pallas as pl
from jax.experimental.pallas import tpu as pltpu
from jax.experimental.pallas import tpu_sc as plsc

_EDGE_SP = 512
_FEAT_SP = 256
_NW = 32
_K = _EDGE_SP + _FEAT_SP
_PPW = _K // _NW
_CH = 192
_CHUNKS = _PPW * _CH // 128


def _sc_gather_body(sar_hbm, rgb_hbm, idx_hbm, sar_out, rgb_out,
                    idx_v, bs_v, br_v, sem_s, sem_r):
    cid = lax.axis_index("c")
    sid = lax.axis_index("s")
    wid = sid * 2 + cid
    pltpu.sync_copy(idx_hbm.at[wid], idx_v)

    def fire(j, carry):
        pltpu.async_copy(sar_hbm.at[idx_v.at[j]], bs_v.at[j], sem_s)
        pltpu.async_copy(rgb_hbm.at[idx_v.at[j]], br_v.at[j], sem_r)
        return carry

    lax.fori_loop(0, _CHUNKS, fire, 0)
    pltpu.make_async_copy(sar_out.at[wid], bs_v, sem_s).wait()
    pltpu.make_async_copy(rgb_out.at[wid], br_v, sem_r).wait()
    pltpu.sync_copy(bs_v, sar_out.at[wid])
    pltpu.sync_copy(br_v, rgb_out.at[wid])


def _vmask_kernel(sar_ref, out_ref):
    out_ref[...] = jnp.sum(sar_ref[...], axis=1)


def _cos_kernel(sa_ref, st_ref, ra_ref, rt_ref, rgb_out, sar_out):
    def cs(a, t):
        num = jax.lax.dot_general(
            a, t, (((1,), (1,)), ((), ())),
            precision=jax.lax.Precision.HIGHEST,
            preferred_element_type=jnp.float32,
        )
        na = jnp.sqrt(jnp.sum(a * a, axis=1, keepdims=True))
        nt = jnp.sqrt(jnp.sum(t * t, axis=1, keepdims=True))
        den = na * nt.T
        return num / jnp.maximum(den, 1e-8)

    rgb_out[...] = cs(ra_ref[...], rt_ref[...])
    sar_out[...] = cs(sa_ref[...], st_ref[...])


def _resize_bilinear(x, out_h, out_w):
    N, C, H, W = x.shape
    ys = jnp.linspace(0.0, H - 1, out_h)
    xs = jnp.linspace(0.0, W - 1, out_w)
    y0 = jnp.floor(ys).astype(jnp.int32)
    y1 = jnp.minimum(y0 + 1, H - 1)
    wy = ys - y0.astype(jnp.float32)
    x0 = jnp.floor(xs).astype(jnp.int32)
    x1 = jnp.minimum(x0 + 1, W - 1)
    wx = xs - x0.astype(jnp.float32)
    rows0 = x[:, :, y0, :]
    rows1 = x[:, :, y1, :]
    top = rows0[:, :, :, x0] * (1.0 - wx)[None, None, None, :] + rows0[:, :, :, x1] * wx[None, None, None, :]
    bot = rows1[:, :, :, x0] * (1.0 - wx)[None, None, None, :] + rows1[:, :, :, x1] * wx[None, None, None, :]
    return top * (1.0 - wy)[None, None, :, None] + bot * wy[None, None, :, None]


def _gumbel_topk(key, weights, k):
    g = jax.random.gumbel(key, weights.shape, dtype=jnp.float32)
    logits = jnp.where(weights > 0, jnp.log(jnp.maximum(weights, 1e-30)), -jnp.inf) + g
    _, idx = jax.lax.approx_max_k(logits, k, recall_target=1.0)
    return idx


def kernel(RGB_feat, SAR_feat, edge_mask):
    N, C, H, W = SAR_feat.shape
    P = N * H * W

    vm = pl.pallas_call(
        _vmask_kernel,
        grid=(N, H // 8),
        in_specs=[pl.BlockSpec((1, C, 8, W), lambda n, h: (n, 0, h, 0))],
        out_specs=pl.BlockSpec((1, 8, W), lambda n, h: (n, h, 0)),
        out_shape=jax.ShapeDtypeStruct((N, H, W), jnp.float32),
    )(SAR_feat)
    valid_mask = (vm.reshape(-1) != 0).astype(jnp.float32)
    feat_weight = valid_mask

    skey = jax.random.key(42)
    kf, ke = jax.random.split(skey)
    feat_index = _gumbel_topk(kf, feat_weight, _FEAT_SP)

    em = _resize_bilinear(edge_mask, H, W)
    em = (em > 0.5).astype(jnp.float32)
    em_flat = jnp.transpose(em, (0, 2, 3, 1)).reshape(-1) * valid_mask
    edge_weight = feat_weight * em_flat
    edge_index = _gumbel_topk(ke, edge_weight, _EDGE_SP)

    all_idx = jnp.concatenate([edge_index, feat_index])
    n_i = all_idx // (H * W)
    hw_i = all_idx % (H * W)
    word_base = n_i * (C * H * W) + hw_i
    widx = word_base[:, None] + (jnp.arange(C, dtype=jnp.int32) * (H * W))[None, :]
    widx = widx.reshape(_NW, _CHUNKS, 128)

    sc_gather = functools.partial(
        pl.kernel,
        out_type=(
            jax.ShapeDtypeStruct((_NW, _CHUNKS, 128), jnp.float32),
            jax.ShapeDtypeStruct((_NW, _CHUNKS, 128), jnp.float32),
        ),
        mesh=plsc.VectorSubcoreMesh(core_axis_name="c", subcore_axis_name="s"),
        scratch_types=[
            pltpu.VMEM((_CHUNKS, 128), jnp.int32),
            pltpu.VMEM((_CHUNKS, 128), jnp.float32),
            pltpu.VMEM((_CHUNKS, 128), jnp.float32),
            pltpu.SemaphoreType.DMA,
            pltpu.SemaphoreType.DMA,
        ],
    )(_sc_gather_body)
    sar_rows, rgb_rows = sc_gather(SAR_feat.reshape(-1), RGB_feat.reshape(-1), widx)
    sar_rows = sar_rows.reshape(_K, C)
    rgb_rows = rgb_rows.reshape(_K, C)
    SAR_anch, SAR_tar = sar_rows[:_EDGE_SP], sar_rows[_EDGE_SP:]
    RGB_anch, RGB_tar = rgb_rows[:_EDGE_SP], rgb_rows[_EDGE_SP:]

    RGB_mat, SAR_mat = pl.pallas_call(
        _cos_kernel,
        out_shape=(
            jax.ShapeDtypeStruct((_EDGE_SP, _FEAT_SP), jnp.float32),
            jax.ShapeDtypeStruct((_EDGE_SP, _FEAT_SP), jnp.float32),
        ),
    )(SAR_anch, SAR_tar, RGB_anch, RGB_tar)
    return (RGB_mat, SAR_mat)
